# Initial kernel scaffold; baseline (speedup 1.0000x reference)
import jax
import jax.numpy as jnp
from jax.experimental import pallas as pl


def kernel(births, deaths, W, bias, k):
    raise NotImplementedError("write your pallas kernel here")


# speedup vs baseline: 3.1882x; 3.1882x over previous
import functools

import jax
import jax.numpy as jnp
from jax.experimental import pallas as pl

_L = 128
_N = 32768
_K = 50
_ROWS = 8


def _stats_kernel(b_ref, d_ref, feats_ref):
    b = b_ref[...]
    d = d_ref[...]
    p = jnp.abs(d - b)
    pb = jax.lax.bitcast_convert_type(p, jnp.int32)
    hi = jnp.max(pb, axis=1, keepdims=True)
    lo = jnp.zeros_like(hi)

    def body(_, carry):
        lo, hi = carry
        mid = lo + (hi - lo + 1) // 2
        cnt = jnp.sum((pb >= mid).astype(jnp.int32), axis=1, keepdims=True)
        pred = cnt >= _K
        return jnp.where(pred, mid, lo), jnp.where(pred, hi, mid - 1)

    lo, hi = jax.lax.fori_loop(0, 31, body, (lo, hi))
    t = jax.lax.bitcast_convert_type(lo, jnp.float32)

    gt = (p > t).astype(jnp.float32)
    eqm = p == t
    cnt_gt = jnp.sum(gt, axis=1, keepdims=True)
    cnt_eq = jnp.sum(eqm.astype(jnp.float32), axis=1, keepdims=True)
    need = jnp.float32(_K) - cnt_gt
    w = gt + jnp.where(eqm, need / cnt_eq, 0.0)

    logp = jnp.log1p(p)
    e5 = jnp.where((w > 0.0) & (p > 0.0), jnp.exp(p - 1.0) * w, 0.0)
    pw = p * w
    f0 = jnp.sum(pw, axis=1)
    f1 = jnp.sum(b * pw, axis=1)
    f2 = jnp.sum(d * pw, axis=1)
    lw = logp * w
    f3 = jnp.sum(b * lw, axis=1)
    f4 = jnp.sum(d * lw, axis=1)
    s5 = jnp.sum(e5, axis=1)
    f5 = jnp.log(jnp.exp(jnp.float32(-1.0)) + s5) + 1.0
    feats_ref[...] = jnp.stack([f0, f1, f2, f3, f4, f5], axis=1)


def _final_kernel(f_ref, w_ref, bias_ref, o_ref):
    feats = f_ref[...]
    mean = jnp.mean(feats, axis=0, keepdims=True)
    var = jnp.mean((feats - mean) ** 2, axis=0, keepdims=True)
    normed = (feats - mean) / jnp.sqrt(var + 1e-5)
    w = w_ref[...]
    out = jnp.sum(normed * w, axis=1, keepdims=True) + bias_ref[0, 0]
    o_ref[...] = out


@jax.jit
def _run(births, deaths, W, bias):
    nrows = births.shape[0]
    feats = pl.pallas_call(
        _stats_kernel,
        grid=(nrows // _ROWS,),
        in_specs=[
            pl.BlockSpec((_ROWS, _N), lambda i: (i, 0)),
            pl.BlockSpec((_ROWS, _N), lambda i: (i, 0)),
        ],
        out_specs=pl.BlockSpec((_ROWS, 6), lambda i: (i, 0)),
        out_shape=jax.ShapeDtypeStruct((nrows, 6), jnp.float32),
    )(births, deaths)
    feats = feats.reshape(_L, 18)
    out = pl.pallas_call(
        _final_kernel,
        in_specs=[
            pl.BlockSpec((_L, 18), lambda: (0, 0)),
            pl.BlockSpec((1, 18), lambda: (0, 0)),
            pl.BlockSpec((1, 1), lambda: (0, 0)),
        ],
        out_specs=pl.BlockSpec((_L, 1), lambda: (0, 0)),
        out_shape=jax.ShapeDtypeStruct((_L, 1), jnp.float32),
    )(feats, W.reshape(1, 18), bias.reshape(1, 1))
    return out


def kernel(births, deaths, W, bias, k):
    return _run(births, deaths, W, bias)
